# Initial kernel scaffold; baseline (speedup 1.0000x reference)
import jax
import jax.numpy as jnp
from jax.experimental import pallas as pl


def kernel(fea, adj_tilde_indices, adj_tilde_values, adj_indices, adj_values, alpha, W, b):
    raise NotImplementedError("write your pallas kernel here")


# speedup vs baseline: 2.8443x; 2.8443x over previous
import functools

import jax
import jax.numpy as jnp
from jax.experimental import pallas as pl
from jax.experimental.pallas import tpu as pltpu
from jax.experimental.pallas import tpu_sc as plsc

_N = 10000
_E = 320000
_F = 128

_K = 128
_NCHUNK = 157
_EPER = _K * _NCHUNK
_EPAD = 16 * _EPER
_NPAD = 10240
_RPT = _NPAD // 16

_mesh = plsc.VectorSubcoreMesh(
    core_axis_name="c", subcore_axis_name="s", num_cores=2, num_subcores=16
)


def _spmm_pair_body(table_hbm, src_hbm, dst_hbm, val_hbm, zeros_hbm, out_hbm,
                    src_v, dst_v, val_v, rows_v, acc_sh, sem):
    cid = jax.lax.axis_index("c")
    sid = jax.lax.axis_index("s")
    rbase = sid * _RPT
    pltpu.sync_copy(zeros_hbm.at[pl.ds(rbase, _RPT)],
                    acc_sh.at[pl.ds(rbase, _RPT)])
    plsc.subcore_barrier()

    ebase = cid * _EPAD + sid * _EPER

    @pl.loop(0, _NCHUNK)
    def _chunk(i):
        b = ebase + i * _K
        pltpu.sync_copy(src_hbm.at[pl.ds(b, _K)], src_v)
        pltpu.sync_copy(dst_hbm.at[pl.ds(b, _K)], dst_v)
        pltpu.sync_copy(val_hbm.at[pl.ds(b, _K)], val_v)
        pltpu.async_copy(table_hbm.at[src_v], rows_v, sem).wait()

        @pl.loop(0, _K)
        def _edge(e):
            v = plsc.load_gather(val_v, [jnp.full((16,), e, jnp.int32)])
            for c in range(8):
                sl = pl.ds(c * 16, 16)
                rows_v[e, sl] = rows_v[e, sl] * v

        pltpu.sync_copy(rows_v, acc_sh.at[dst_v], add=True)

    plsc.subcore_barrier()
    pltpu.sync_copy(acc_sh.at[pl.ds(rbase, _RPT)],
                    out_hbm.at[cid, pl.ds(rbase, _RPT)])


_spmm_pair = functools.partial(
    pl.kernel,
    out_type=jax.ShapeDtypeStruct((2, _NPAD, _F), jnp.float32),
    mesh=_mesh,
    compiler_params=pltpu.CompilerParams(needs_layout_passes=False),
    scratch_types=[
        pltpu.VMEM((_K,), jnp.int32),
        pltpu.VMEM((_K,), jnp.int32),
        pltpu.VMEM((_K,), jnp.float32),
        pltpu.VMEM((_K, _F), jnp.float32),
        pltpu.VMEM_SHARED((_NPAD, _F), jnp.float32),
        pltpu.SemaphoreType.DMA,
    ],
)(_spmm_pair_body)


_BLK = 1000


def _dense_body(f_ref, x1_ref, x1t_ref, x2_ref, x2t_ref, w_ref, ae_ref, b_ref,
                o_ref):
    h = jnp.concatenate(
        [f_ref[...], x1_ref[...], x1t_ref[...], x2_ref[...], x2t_ref[...]],
        axis=1)
    w = w_ref[...] * ae_ref[...]
    z = jnp.dot(h.astype(jnp.bfloat16), w.astype(jnp.bfloat16),
                preferred_element_type=jnp.float32)
    o_ref[...] = jnp.maximum(z + b_ref[...], 0.0)


def _dense(fea, x1, x1t, x2, x2t, w, alpha_exp, b2):
    row_spec = pl.BlockSpec((_BLK, _F), lambda i: (i, 0))
    return pl.pallas_call(
        _dense_body,
        grid=(_N // _BLK,),
        in_specs=[
            row_spec, row_spec, row_spec, row_spec, row_spec,
            pl.BlockSpec((5 * _F, _F), lambda i: (0, 0)),
            pl.BlockSpec((5 * _F, 1), lambda i: (0, 0)),
            pl.BlockSpec((1, _F), lambda i: (0, 0)),
        ],
        out_specs=row_spec,
        out_shape=jax.ShapeDtypeStruct((_N, _F), jnp.float32),
    )(fea, x1, x1t, x2, x2t, w, alpha_exp, b2)


def kernel(fea, adj_tilde_indices, adj_tilde_values, adj_indices, adj_values,
           alpha, W, b):
    pad = _EPAD - _E
    i32 = jnp.int32
    src_all = jnp.concatenate([
        adj_indices[1].astype(i32), jnp.zeros((pad,), i32),
        adj_tilde_indices[1].astype(i32) + _NPAD, jnp.full((pad,), _NPAD, i32),
    ])
    dst_all = jnp.concatenate([
        adj_indices[0].astype(i32), jnp.zeros((pad,), i32),
        adj_tilde_indices[0].astype(i32), jnp.zeros((pad,), i32),
    ])
    zpad = jnp.zeros((pad,), jnp.float32)
    val_all = jnp.concatenate([adj_values, zpad, adj_tilde_values, zpad])
    zeros = jnp.zeros((_NPAD, _F), jnp.float32)

    rowpad = jnp.zeros((_NPAD - _N, _F), jnp.float32)
    table1 = jnp.concatenate([fea, rowpad, fea, rowpad], axis=0)
    r1 = _spmm_pair(table1, src_all, dst_all, val_all, zeros)
    r2 = _spmm_pair(r1.reshape(2 * _NPAD, _F), src_all, dst_all, val_all, zeros)

    alpha_exp = jnp.repeat(alpha, _F)[:, None]
    return _dense(fea, r1[0, :_N], r1[1, :_N], r2[0, :_N], r2[1, :_N], W,
                  alpha_exp, b.reshape(1, _F))


# speedup vs baseline: 5.2263x; 1.8375x over previous
import functools

import jax
import jax.numpy as jnp
from jax.experimental import pallas as pl
from jax.experimental.pallas import tpu as pltpu
from jax.experimental.pallas import tpu_sc as plsc

_N = 10000
_E = 320000
_F = 128

_K = 120
_NCHUNK = 168
_EPER = _K * _NCHUNK
_EPAD = 16 * _EPER
_NPAD = 10240
_RPT = _NPAD // 16

_mesh = plsc.VectorSubcoreMesh(
    core_axis_name="c", subcore_axis_name="s", num_cores=2, num_subcores=16
)


def _spmm_pair_body(table_hbm, src_hbm, dst_hbm, val_hbm, zeros_hbm, out_hbm,
                    src_v, dst_v, val_v, rows, acc_sh,
                    g0, g1, g2, s0, s1, s2, p0, p1, p2):
    cid = jax.lax.axis_index("c")
    sid = jax.lax.axis_index("s")
    gsem = (g0, g1, g2)
    ssem = (s0, s1, s2)
    psem = (p0, p1, p2)
    rbase = sid * _RPT
    pltpu.sync_copy(zeros_hbm.at[pl.ds(rbase, _RPT)],
                    acc_sh.at[pl.ds(rbase, _RPT)])

    def start_idx(i, b):
        pltpu.async_copy(src_hbm.at[cid, sid, i], src_v.at[b], psem[b])
        pltpu.async_copy(dst_hbm.at[cid, sid, i], dst_v.at[b], psem[b])
        pltpu.async_copy(val_hbm.at[cid, sid, i], val_v.at[b], psem[b])

    def wait_idx(i, b):
        pltpu.make_async_copy(src_hbm.at[cid, sid, i], src_v.at[b],
                              psem[b]).wait()
        pltpu.make_async_copy(dst_hbm.at[cid, sid, i], dst_v.at[b],
                              psem[b]).wait()
        pltpu.make_async_copy(val_hbm.at[cid, sid, i], val_v.at[b],
                              psem[b]).wait()

    def start_gather(b):
        pltpu.async_copy(table_hbm.at[src_v.at[b]], rows.at[b], gsem[b])

    def wait_gather(b):
        pltpu.make_async_copy(table_hbm.at[src_v.at[b]], rows.at[b],
                              gsem[b]).wait()

    def start_scatter(b):
        pltpu.async_copy(rows.at[b], acc_sh.at[dst_v.at[b]], ssem[b],
                         add=True)

    def wait_scatter(b):
        pltpu.make_async_copy(rows.at[b], acc_sh.at[dst_v.at[b]],
                              ssem[b]).wait()

    def stage(i, b):
        wait_gather(b)
        bn = (b + 1) % 3

        @pl.when(i + 1 < _NCHUNK)
        def _():
            wait_idx(i + 1, bn)
            start_gather(bn)

        @pl.loop(0, _K, unroll=2)
        def _edge(e):
            v = plsc.load_gather(val_v, [jnp.full((16,), b, jnp.int32),
                                         jnp.full((16,), e, jnp.int32)])
            for c in range(8):
                sl = pl.ds(c * 16, 16)
                rows[b, e, sl] = rows[b, e, sl] * v

        start_scatter(b)
        bp = (b + 2) % 3

        @pl.when(i >= 1)
        def _():
            wait_scatter(bp)

        @pl.when(i + 2 < _NCHUNK)
        def _():
            start_idx(i + 2, bp)

    start_idx(0, 0)
    start_idx(1, 1)
    plsc.subcore_barrier()
    wait_idx(0, 0)
    start_gather(0)

    @pl.loop(0, _NCHUNK // 3)
    def _chunk(j):
        i = j * 3
        stage(i, 0)
        stage(i + 1, 1)
        stage(i + 2, 2)

    wait_scatter(2)

    plsc.subcore_barrier()
    pltpu.sync_copy(acc_sh.at[pl.ds(rbase, _RPT)],
                    out_hbm.at[cid, pl.ds(rbase, _RPT)])


_spmm_pair = functools.partial(
    pl.kernel,
    out_type=jax.ShapeDtypeStruct((2, _NPAD, _F), jnp.float32),
    mesh=_mesh,
    compiler_params=pltpu.CompilerParams(needs_layout_passes=False),
    scratch_types=[
        pltpu.VMEM((3, _K), jnp.int32),
        pltpu.VMEM((3, _K), jnp.int32),
        pltpu.VMEM((3, _K), jnp.float32),
        pltpu.VMEM((3, _K, _F), jnp.float32),
        pltpu.VMEM_SHARED((_NPAD, _F), jnp.float32),
    ] + [pltpu.SemaphoreType.DMA] * 9,
)(_spmm_pair_body)


_BLK = 1000


def _dense_body(f_ref, x1_ref, x1t_ref, x2_ref, x2t_ref, w_ref, ae_ref, b_ref,
                o_ref):
    h = jnp.concatenate(
        [f_ref[...], x1_ref[...], x1t_ref[...], x2_ref[...], x2t_ref[...]],
        axis=1)
    w = w_ref[...] * ae_ref[...]
    z = jnp.dot(h.astype(jnp.bfloat16), w.astype(jnp.bfloat16),
                preferred_element_type=jnp.float32)
    o_ref[...] = jnp.maximum(z + b_ref[...], 0.0)


def _dense(fea, x1, x1t, x2, x2t, w, alpha_exp, b2):
    row_spec = pl.BlockSpec((_BLK, _F), lambda i: (i, 0))
    return pl.pallas_call(
        _dense_body,
        grid=(_N // _BLK,),
        in_specs=[
            row_spec, row_spec, row_spec, row_spec, row_spec,
            pl.BlockSpec((5 * _F, _F), lambda i: (0, 0)),
            pl.BlockSpec((5 * _F, 1), lambda i: (0, 0)),
            pl.BlockSpec((1, _F), lambda i: (0, 0)),
        ],
        out_specs=row_spec,
        out_shape=jax.ShapeDtypeStruct((_N, _F), jnp.float32),
    )(fea, x1, x1t, x2, x2t, w, alpha_exp, b2)


def kernel(fea, adj_tilde_indices, adj_tilde_values, adj_indices, adj_values,
           alpha, W, b):
    pad = _EPAD - _E
    i32 = jnp.int32
    shp = (2, 16, _NCHUNK, _K)
    src_all = jnp.concatenate([
        adj_indices[1].astype(i32), jnp.zeros((pad,), i32),
        adj_tilde_indices[1].astype(i32) + _NPAD, jnp.full((pad,), _NPAD, i32),
    ]).reshape(shp)
    dst_all = jnp.concatenate([
        adj_indices[0].astype(i32), jnp.zeros((pad,), i32),
        adj_tilde_indices[0].astype(i32), jnp.zeros((pad,), i32),
    ]).reshape(shp)
    zpad = jnp.zeros((pad,), jnp.float32)
    val_all = jnp.concatenate(
        [adj_values, zpad, adj_tilde_values, zpad]).reshape(shp)
    zeros = jnp.zeros((_NPAD, _F), jnp.float32)

    rowpad = jnp.zeros((_NPAD - _N, _F), jnp.float32)
    table1 = jnp.concatenate([fea, rowpad, fea, rowpad], axis=0)
    r1 = _spmm_pair(table1, src_all, dst_all, val_all, zeros)
    r2 = _spmm_pair(r1.reshape(2 * _NPAD, _F), src_all, dst_all, val_all,
                    zeros)

    alpha_exp = jnp.repeat(alpha, _F)[:, None]
    return _dense(fea, r1[0, :_N], r1[1, :_N], r2[0, :_N], r2[1, :_N], W,
                  alpha_exp, b.reshape(1, _F))
